# Initial kernel scaffold; baseline (speedup 1.0000x reference)
import functools

import jax
import jax.numpy as jnp
from jax import lax
from jax.experimental import pallas as pl
from jax.experimental.pallas import tpu as pltpu
from jax.experimental.pallas import tpu_sc as plsc

N_NODES = 10000
N_EDGES = 320000
N_REL = 16
NR = N_NODES * N_REL
NR_PAD = 163840
D = 128
K = 128
NCHUNKS = N_EDGES // K
NW = 32
CHUNK_BASE = NCHUNKS // NW

_f32 = jnp.float32
_i32 = jnp.int32

_MESH = plsc.VectorSubcoreMesh(
    core_axis_name="c", subcore_axis_name="s", num_cores=2, num_subcores=16)

_GATHER_DNUMS = lax.GatherDimensionNumbers(
    offset_dims=(), collapsed_slice_dims=(0,), start_index_map=(0,))


def _bcast_lane(vec16, j):
  idx = jnp.full((16, 1), j, _i32)
  return lax.gather(vec16, idx, dimension_numbers=_GATHER_DNUMS,
                    slice_sizes=(1,),
                    mode=lax.GatherScatterMode.PROMISE_IN_BOUNDS)


def _worker_id():
  return lax.axis_index("s") * 2 + lax.axis_index("c")


def _nchunks(wid):
  return CHUNK_BASE + jnp.where(wid < (NCHUNKS - CHUNK_BASE * NW), 1, 0)


@functools.partial(
    pl.kernel,
    out_type=(jax.ShapeDtypeStruct((2, NR_PAD), _f32),
              jax.ShapeDtypeStruct((N_EDGES,), _i32),
              jax.ShapeDtypeStruct((N_EDGES,), _i32)),
    mesh=_MESH,
    scratch_types=[
        pltpu.VMEM((K,), _i32),
        pltpu.VMEM((K,), _i32),
        pltpu.VMEM((K,), _i32),
        pltpu.VMEM((K,), _i32),
        pltpu.VMEM((K,), _i32),
        pltpu.VMEM((K,), _f32),
        pltpu.VMEM_SHARED((NR_PAD,), _f32),
    ],
)
def _counts_kernel(ei_hbm, et_hbm, zc_hbm,
                   cnt_out, gidx_out, cidx_out,
                   src_v, dst_v, typ_v, gi_v, ci_v, ones_v, acc_sh):
  cid = lax.axis_index("c")
  sid = lax.axis_index("s")
  wid = _worker_id()

  for k in range(K // 16):
    ones_v[pl.ds(k * 16, 16)] = jnp.full((16,), 1.0, _f32)

  seg = NR_PAD // 16
  pltpu.sync_copy(zc_hbm.at[pl.ds(sid * seg, seg)],
                  acc_sh.at[pl.ds(sid * seg, seg)])
  plsc.subcore_barrier()

  def chunk_body(i, carry):
    off = (wid + NW * i) * K
    pltpu.sync_copy(ei_hbm.at[0, pl.ds(off, K)], src_v)
    pltpu.sync_copy(ei_hbm.at[1, pl.ds(off, K)], dst_v)
    pltpu.sync_copy(et_hbm.at[pl.ds(off, K)], typ_v)
    for k in range(K // 16):
      s16 = src_v[pl.ds(k * 16, 16)]
      d16 = dst_v[pl.ds(k * 16, 16)]
      t16 = typ_v[pl.ds(k * 16, 16)]
      gi_v[pl.ds(k * 16, 16)] = t16 * N_NODES + s16
      ci_v[pl.ds(k * 16, 16)] = d16 * N_REL + t16
    pltpu.sync_copy(gi_v, gidx_out.at[pl.ds(off, K)])
    pltpu.sync_copy(ci_v, cidx_out.at[pl.ds(off, K)])
    pltpu.sync_copy(ones_v, acc_sh.at[ci_v], add=True)
    return carry

  lax.fori_loop(0, _nchunks(wid), chunk_body, 0)
  plsc.subcore_barrier()
  pltpu.sync_copy(acc_sh.at[pl.ds(sid * seg, seg)],
                  cnt_out.at[cid, pl.ds(sid * seg, seg)])


@functools.partial(
    pl.kernel,
    out_type=jax.ShapeDtypeStruct((2, N_NODES, D), _f32),
    mesh=_MESH,
    scratch_types=[
        pltpu.VMEM((K,), _i32),
        pltpu.VMEM((K,), _i32),
        pltpu.VMEM((K,), _i32),
        pltpu.VMEM((K,), _f32),
        pltpu.VMEM((K, D), _f32),
        pltpu.SemaphoreType.DMA,
        pltpu.VMEM_SHARED((N_NODES, D), _f32),
        pltpu.VMEM_SHARED((NR_PAD,), _f32),
    ],
)
def _agg_kernel(hx_hbm, ei_hbm, gidx_hbm, cidx_hbm, recip_hbm, za_hbm,
                out_hbm,
                gi_v, ci_v, d_v, w_v, rows_v, sem, acc_sh, recip_sh):
  cid = lax.axis_index("c")
  sid = lax.axis_index("s")
  wid = _worker_id()

  rows_per_tile = N_NODES // 16
  pltpu.sync_copy(za_hbm.at[pl.ds(sid * rows_per_tile, rows_per_tile), :],
                  acc_sh.at[pl.ds(sid * rows_per_tile, rows_per_tile), :])
  seg = NR_PAD // 16
  pltpu.sync_copy(recip_hbm.at[pl.ds(sid * seg, seg)],
                  recip_sh.at[pl.ds(sid * seg, seg)])
  plsc.subcore_barrier()

  def chunk_body(i, carry):
    off = (wid + NW * i) * K
    pltpu.sync_copy(gidx_hbm.at[pl.ds(off, K)], gi_v)
    pltpu.sync_copy(cidx_hbm.at[pl.ds(off, K)], ci_v)
    pltpu.sync_copy(ei_hbm.at[1, pl.ds(off, K)], d_v)
    pltpu.async_copy(hx_hbm.at[gi_v], rows_v, sem).wait()
    pltpu.sync_copy(recip_sh.at[ci_v], w_v)

    def scale_group(g, c2):
      w16 = w_v[pl.ds(g * 16, 16)]
      for j in range(16):
        wb = _bcast_lane(w16, j)
        e = g * 16 + j
        for k in range(D // 16):
          rows_v[e, pl.ds(k * 16, 16)] = rows_v[e, pl.ds(k * 16, 16)] * wb
      return c2

    lax.fori_loop(0, K // 16, scale_group, 0)
    pltpu.sync_copy(rows_v, acc_sh.at[d_v], add=True)
    return carry

  lax.fori_loop(0, _nchunks(wid), chunk_body, 0)
  plsc.subcore_barrier()
  pltpu.sync_copy(acc_sh.at[pl.ds(sid * rows_per_tile, rows_per_tile), :],
                  out_hbm.at[cid, pl.ds(sid * rows_per_tile, rows_per_tile), :])


def _combine_body(comp_ref, basis_ref, out_ref):
  out_ref[...] = jnp.dot(comp_ref[...], basis_ref[...],
                         preferred_element_type=_f32)


def _combine(comp, basis_flat):
  nb = comp.shape[1]
  f = basis_flat.shape[1]
  blk = 2048
  return pl.pallas_call(
      _combine_body,
      grid=(f // blk,),
      in_specs=[
          pl.BlockSpec((N_REL, nb), lambda i: (0, 0)),
          pl.BlockSpec((nb, blk), lambda i: (0, i)),
      ],
      out_specs=pl.BlockSpec((N_REL, blk), lambda i: (0, i)),
      out_shape=jax.ShapeDtypeStruct((N_REL, f), _f32),
  )(comp, basis_flat)


def _hx_body(x_ref, w_ref, out_ref):
  out_ref[0] = jnp.dot(x_ref[...], w_ref[0], preferred_element_type=_f32)


def _hx(x, w_full):
  bn = 1000
  din = x.shape[1]
  return pl.pallas_call(
      _hx_body,
      grid=(N_NODES // bn, 17),
      in_specs=[
          pl.BlockSpec((bn, din), lambda nb, r: (nb, 0)),
          pl.BlockSpec((1, din, D), lambda nb, r: (r, 0, 0)),
      ],
      out_specs=pl.BlockSpec((1, bn, D), lambda nb, r: (r, nb, 0)),
      out_shape=jax.ShapeDtypeStruct((17, N_NODES, D), _f32),
  )(x, w_full)


def _recip_body(cnt_ref, out_ref):
  c = cnt_ref[0] + cnt_ref[1]
  out_ref[...] = 1.0 / jnp.maximum(c, 1.0)


def _recip(cnt):
  rows = NR_PAD // D
  blk = 128
  out = pl.pallas_call(
      _recip_body,
      grid=(rows // blk,),
      in_specs=[pl.BlockSpec((2, blk, D), lambda i: (0, i, 0))],
      out_specs=pl.BlockSpec((blk, D), lambda i: (i, 0)),
      out_shape=jax.ShapeDtypeStruct((rows, D), _f32),
  )(cnt.reshape(2, rows, D))
  return out.reshape(NR_PAD)


def _z_body(agg_ref, root_ref, bias_ref, out_ref):
  s = agg_ref[0] + agg_ref[1] + root_ref[0] + bias_ref[...]
  out_ref[...] = jnp.where(s >= 0, s, 0.01 * s)


def _z_layer(agg, hx1, bias1):
  bn = 1000
  return pl.pallas_call(
      _z_body,
      grid=(N_NODES // bn,),
      in_specs=[
          pl.BlockSpec((2, bn, D), lambda i: (0, i, 0)),
          pl.BlockSpec((1, bn, D), lambda i: (16, i, 0)),
          pl.BlockSpec((1, D), lambda i: (0, 0)),
      ],
      out_specs=pl.BlockSpec((bn, D), lambda i: (i, 0)),
      out_shape=jax.ShapeDtypeStruct((N_NODES, D), _f32),
  )(agg, hx1, bias1.reshape(1, D))


def _final_body(agg_ref, root_ref, bias_ref, mu_ref, ls_ref):
  s = agg_ref[0] + agg_ref[1] + root_ref[0] + bias_ref[...]
  mu_ref[...] = s[:, :64]
  ls_ref[...] = s[:, 64:]


def _final(agg23, hx23, bias23):
  bn = 1000
  return pl.pallas_call(
      _final_body,
      grid=(N_NODES // bn,),
      in_specs=[
          pl.BlockSpec((2, bn, D), lambda i: (0, i, 0)),
          pl.BlockSpec((1, bn, D), lambda i: (16, i, 0)),
          pl.BlockSpec((1, D), lambda i: (0, 0)),
      ],
      out_specs=[
          pl.BlockSpec((bn, 64), lambda i: (i, 0)),
          pl.BlockSpec((bn, 64), lambda i: (i, 0)),
      ],
      out_shape=(jax.ShapeDtypeStruct((N_NODES, 64), _f32),
                 jax.ShapeDtypeStruct((N_NODES, 64), _f32)),
  )(agg23, hx23, bias23.reshape(1, D))


def kernel(x, edge_index, edge_type, comp1, basis1, root1, bias1,
           comp_mu, basis_mu, root_mu, bias_mu,
           comp_ls, basis_ls, root_ls, bias_ls):
  nb = basis1.shape[0]

  zc = jnp.zeros((NR_PAD,), _f32)
  za = jnp.zeros((N_NODES, D), _f32)

  cnt, gidx, cidx = _counts_kernel(edge_index, edge_type, zc)
  recip = _recip(cnt)

  w1 = _combine(comp1, basis1.reshape(nb, -1)).reshape(N_REL, D, D)
  w1_full = jnp.concatenate([w1, root1[None]], axis=0)
  hx1 = _hx(x, w1_full)
  agg1 = _agg_kernel(hx1.reshape(17 * N_NODES, D), edge_index, gidx, cidx,
                     recip, za)
  z = _z_layer(agg1, hx1, bias1)

  wmu = _combine(comp_mu, basis_mu.reshape(nb, -1)).reshape(N_REL, D, 64)
  wls = _combine(comp_ls, basis_ls.reshape(nb, -1)).reshape(N_REL, D, 64)
  w23 = jnp.concatenate([wmu, wls], axis=2)
  root23 = jnp.concatenate([root_mu, root_ls], axis=1)
  w23_full = jnp.concatenate([w23, root23[None]], axis=0)
  hx23 = _hx(z, w23_full)
  agg23 = _agg_kernel(hx23.reshape(17 * N_NODES, D), edge_index, gidx, cidx,
                      recip, za)
  bias23 = jnp.concatenate([bias_mu, bias_ls], axis=0)
  mu, logstd = _final(agg23, hx23, bias23)
  return (mu, logstd)


# speedup vs baseline: 13.0900x; 13.0900x over previous
import functools

import jax
import jax.numpy as jnp
from jax import lax
from jax.experimental import pallas as pl
from jax.experimental.pallas import tpu as pltpu
from jax.experimental.pallas import tpu_sc as plsc

N_NODES = 10000
N_EDGES = 320000
N_REL = 16
NR = N_NODES * N_REL
NR_PAD = 163840
D = 128
K = 128
NCHUNKS = N_EDGES // K
NW = 32
CHUNK_BASE = NCHUNKS // NW
N_ACC = 10240
RPT = N_ACC // 16

_f32 = jnp.float32
_i32 = jnp.int32

_MESH = plsc.VectorSubcoreMesh(
    core_axis_name="c", subcore_axis_name="s", num_cores=2, num_subcores=16)

_GATHER_DNUMS = lax.GatherDimensionNumbers(
    offset_dims=(), collapsed_slice_dims=(0,), start_index_map=(0,))


def _bcast_lane(vec16, j):
  idx = jnp.full((16, 1), j, _i32)
  return lax.gather(vec16, idx, dimension_numbers=_GATHER_DNUMS,
                    slice_sizes=(1,),
                    mode=lax.GatherScatterMode.PROMISE_IN_BOUNDS)


def _worker_id():
  return lax.axis_index("s") * 2 + lax.axis_index("c")


def _nchunks(wid):
  return CHUNK_BASE + jnp.where(wid < (NCHUNKS - CHUNK_BASE * NW), 1, 0)


@functools.partial(
    pl.kernel,
    out_type=(jax.ShapeDtypeStruct((2 * NR_PAD,), _f32),
              jax.ShapeDtypeStruct((N_EDGES,), _i32),
              jax.ShapeDtypeStruct((N_EDGES,), _i32)),
    mesh=_MESH,
    scratch_types=[
        pltpu.VMEM((K,), _i32),
        pltpu.VMEM((K,), _i32),
        pltpu.VMEM((K,), _i32),
        pltpu.VMEM((K,), _i32),
        pltpu.VMEM((K,), _i32),
        pltpu.VMEM((K,), _f32),
        pltpu.VMEM_SHARED((NR_PAD,), _f32),
    ],
)
def _counts_kernel(src_hbm, dst_hbm, et_hbm, zc_hbm,
                   cnt_out, gidx_out, cidx_out,
                   src_v, dst_v, typ_v, gi_v, ci_v, ones_v, acc_sh):
  cid = lax.axis_index("c")
  sid = lax.axis_index("s")
  wid = _worker_id()

  for k in range(K // 16):
    ones_v[pl.ds(k * 16, 16)] = jnp.full((16,), 1.0, _f32)

  seg = NR_PAD // 16
  pltpu.sync_copy(zc_hbm.at[pl.ds(sid * seg, seg)],
                  acc_sh.at[pl.ds(sid * seg, seg)])
  plsc.subcore_barrier()

  def chunk_body(i, carry):
    off = (wid + NW * i) * K
    pltpu.sync_copy(src_hbm.at[pl.ds(off, K)], src_v)
    pltpu.sync_copy(dst_hbm.at[pl.ds(off, K)], dst_v)
    pltpu.sync_copy(et_hbm.at[pl.ds(off, K)], typ_v)
    for k in range(K // 16):
      s16 = src_v[pl.ds(k * 16, 16)]
      d16 = dst_v[pl.ds(k * 16, 16)]
      t16 = typ_v[pl.ds(k * 16, 16)]
      gi_v[pl.ds(k * 16, 16)] = t16 * N_NODES + s16
      ci_v[pl.ds(k * 16, 16)] = d16 * N_REL + t16
    pltpu.sync_copy(gi_v, gidx_out.at[pl.ds(off, K)])
    pltpu.sync_copy(ci_v, cidx_out.at[pl.ds(off, K)])
    pltpu.sync_copy(ones_v, acc_sh.at[ci_v], add=True)
    return carry

  lax.fori_loop(0, _nchunks(wid), chunk_body, 0)
  plsc.subcore_barrier()
  pltpu.sync_copy(acc_sh.at[pl.ds(sid * seg, seg)],
                  cnt_out.at[pl.ds(cid * NR_PAD + sid * seg, seg)])


@functools.partial(
    pl.kernel,
    out_type=jax.ShapeDtypeStruct((2, N_ACC, D), _f32),
    mesh=_MESH,
    scratch_types=[
        pltpu.VMEM((K,), _i32),
        pltpu.VMEM((K,), _i32),
        pltpu.VMEM((K,), _i32),
        pltpu.VMEM((K,), _f32),
        pltpu.VMEM((K, D), _f32),
        pltpu.SemaphoreType.DMA,
        pltpu.VMEM_SHARED((N_ACC, D), _f32),
        pltpu.VMEM_SHARED((NR_PAD,), _f32),
    ],
)
def _agg_kernel(hx_hbm, dst_hbm, gidx_hbm, cidx_hbm, recip_hbm, za_hbm,
                out_hbm,
                gi_v, ci_v, d_v, w_v, rows_v, sem, acc_sh, recip_sh):
  cid = lax.axis_index("c")
  sid = lax.axis_index("s")
  wid = _worker_id()

  pltpu.sync_copy(za_hbm.at[pl.ds(sid * RPT, RPT), :],
                  acc_sh.at[pl.ds(sid * RPT, RPT), :])
  seg = NR_PAD // 16
  pltpu.sync_copy(recip_hbm.at[pl.ds(sid * seg, seg)],
                  recip_sh.at[pl.ds(sid * seg, seg)])
  plsc.subcore_barrier()

  def chunk_body(i, carry):
    off = (wid + NW * i) * K
    pltpu.sync_copy(gidx_hbm.at[pl.ds(off, K)], gi_v)
    pltpu.sync_copy(cidx_hbm.at[pl.ds(off, K)], ci_v)
    pltpu.sync_copy(dst_hbm.at[pl.ds(off, K)], d_v)
    pltpu.async_copy(hx_hbm.at[gi_v], rows_v, sem).wait()
    pltpu.sync_copy(recip_sh.at[ci_v], w_v)

    def scale_group(g, c2):
      w16 = w_v[pl.ds(g * 16, 16)]
      for j in range(16):
        wb = _bcast_lane(w16, j)
        e = g * 16 + j
        for k in range(D // 16):
          rows_v[e, pl.ds(k * 16, 16)] = rows_v[e, pl.ds(k * 16, 16)] * wb
      return c2

    lax.fori_loop(0, K // 16, scale_group, 0)
    pltpu.sync_copy(rows_v, acc_sh.at[d_v], add=True)
    return carry

  lax.fori_loop(0, _nchunks(wid), chunk_body, 0)
  plsc.subcore_barrier()
  pltpu.sync_copy(acc_sh.at[pl.ds(sid * RPT, RPT), :],
                  out_hbm.at[cid, pl.ds(sid * RPT, RPT), :])


def _combine_body(comp_ref, basis_ref, out_ref):
  out_ref[...] = jnp.dot(comp_ref[...], basis_ref[...],
                         preferred_element_type=_f32)


def _combine(comp, basis_flat):
  nb = comp.shape[1]
  f = basis_flat.shape[1]
  blk = 2048
  return pl.pallas_call(
      _combine_body,
      grid=(f // blk,),
      in_specs=[
          pl.BlockSpec((N_REL, nb), lambda i: (0, 0)),
          pl.BlockSpec((nb, blk), lambda i: (0, i)),
      ],
      out_specs=pl.BlockSpec((N_REL, blk), lambda i: (0, i)),
      out_shape=jax.ShapeDtypeStruct((N_REL, f), _f32),
  )(comp, basis_flat)


def _hx_body(x_ref, w_ref, out_ref):
  out_ref[0] = jnp.dot(x_ref[...], w_ref[0], preferred_element_type=_f32)


def _hx(x, w_full):
  bn = 1000
  din = x.shape[1]
  return pl.pallas_call(
      _hx_body,
      grid=(N_NODES // bn, 17),
      in_specs=[
          pl.BlockSpec((bn, din), lambda nb, r: (nb, 0)),
          pl.BlockSpec((1, din, D), lambda nb, r: (r, 0, 0)),
      ],
      out_specs=pl.BlockSpec((1, bn, D), lambda nb, r: (r, nb, 0)),
      out_shape=jax.ShapeDtypeStruct((17, N_NODES, D), _f32),
  )(x, w_full)


def _recip_body(cnt_ref, out_ref):
  c = cnt_ref[0] + cnt_ref[1]
  out_ref[...] = 1.0 / jnp.maximum(c, 1.0)


def _recip(cnt):
  rows = NR_PAD // D
  blk = 128
  out = pl.pallas_call(
      _recip_body,
      grid=(rows // blk,),
      in_specs=[pl.BlockSpec((2, blk, D), lambda i: (0, i, 0))],
      out_specs=pl.BlockSpec((blk, D), lambda i: (i, 0)),
      out_shape=jax.ShapeDtypeStruct((rows, D), _f32),
  )(cnt.reshape(2, rows, D))
  return out.reshape(NR_PAD)


def _z_body(agg_ref, root_ref, bias_ref, out_ref):
  s = agg_ref[0] + agg_ref[1] + root_ref[0] + bias_ref[...]
  out_ref[...] = jnp.where(s >= 0, s, 0.01 * s)


def _z_layer(agg, hx1, bias1):
  bn = 1000
  return pl.pallas_call(
      _z_body,
      grid=(N_NODES // bn,),
      in_specs=[
          pl.BlockSpec((2, bn, D), lambda i: (0, i, 0)),
          pl.BlockSpec((1, bn, D), lambda i: (16, i, 0)),
          pl.BlockSpec((1, D), lambda i: (0, 0)),
      ],
      out_specs=pl.BlockSpec((bn, D), lambda i: (i, 0)),
      out_shape=jax.ShapeDtypeStruct((N_NODES, D), _f32),
  )(agg, hx1, bias1.reshape(1, D))


def _final_body(agg_ref, root_ref, bias_ref, mu_ref, ls_ref):
  s = agg_ref[0] + agg_ref[1] + root_ref[0] + bias_ref[...]
  mu_ref[...] = s[:, :64]
  ls_ref[...] = s[:, 64:]


def _final(agg23, hx23, bias23):
  bn = 1000
  return pl.pallas_call(
      _final_body,
      grid=(N_NODES // bn,),
      in_specs=[
          pl.BlockSpec((2, bn, D), lambda i: (0, i, 0)),
          pl.BlockSpec((1, bn, D), lambda i: (16, i, 0)),
          pl.BlockSpec((1, D), lambda i: (0, 0)),
      ],
      out_specs=[
          pl.BlockSpec((bn, 64), lambda i: (i, 0)),
          pl.BlockSpec((bn, 64), lambda i: (i, 0)),
      ],
      out_shape=(jax.ShapeDtypeStruct((N_NODES, 64), _f32),
                 jax.ShapeDtypeStruct((N_NODES, 64), _f32)),
  )(agg23, hx23, bias23.reshape(1, D))


def kernel(x, edge_index, edge_type, comp1, basis1, root1, bias1,
           comp_mu, basis_mu, root_mu, bias_mu,
           comp_ls, basis_ls, root_ls, bias_ls):
  nb = basis1.shape[0]

  zc = jnp.zeros((NR_PAD,), _f32)
  za = jnp.zeros((N_ACC, D), _f32)
  src_arr = edge_index[0]
  dst_arr = edge_index[1]

  cnt, gidx, cidx = _counts_kernel(src_arr, dst_arr, edge_type, zc)
  recip = _recip(cnt)

  w1 = _combine(comp1, basis1.reshape(nb, -1)).reshape(N_REL, D, D)
  w1_full = jnp.concatenate([w1, root1[None]], axis=0)
  hx1 = _hx(x, w1_full)
  agg1 = _agg_kernel(hx1.reshape(17 * N_NODES, D), dst_arr, gidx, cidx,
                     recip, za)
  z = _z_layer(agg1, hx1, bias1)

  wmu = _combine(comp_mu, basis_mu.reshape(nb, -1)).reshape(N_REL, D, 64)
  wls = _combine(comp_ls, basis_ls.reshape(nb, -1)).reshape(N_REL, D, 64)
  w23 = jnp.concatenate([wmu, wls], axis=2)
  root23 = jnp.concatenate([root_mu, root_ls], axis=1)
  w23_full = jnp.concatenate([w23, root23[None]], axis=0)
  hx23 = _hx(z, w23_full)
  agg23 = _agg_kernel(hx23.reshape(17 * N_NODES, D), dst_arr, gidx, cidx,
                      recip, za)
  bias23 = jnp.concatenate([bias_mu, bias_ls], axis=0)
  mu, logstd = _final(agg23, hx23, bias23)
  return (mu, logstd)


# speedup vs baseline: 13.4663x; 1.0287x over previous
import functools

import jax
import jax.numpy as jnp
from jax import lax
from jax.experimental import pallas as pl
from jax.experimental.pallas import tpu as pltpu
from jax.experimental.pallas import tpu_sc as plsc

N_NODES = 10000
N_EDGES = 320000
N_REL = 16
NR = N_NODES * N_REL
NR_PAD = 163840
D = 128
K = 128
NCHUNKS = N_EDGES // K
NW = 32
N_ACC = 10240
RPT = N_ACC // 16
CPT = 80

_f32 = jnp.float32
_i32 = jnp.int32

_MESH = plsc.VectorSubcoreMesh(
    core_axis_name="c", subcore_axis_name="s", num_cores=2, num_subcores=16)

_GATHER_DNUMS = lax.GatherDimensionNumbers(
    offset_dims=(), collapsed_slice_dims=(0,), start_index_map=(0,))


def _bcast_lane(vec16, j):
  idx = jnp.full((16, 1), j, _i32)
  return lax.gather(vec16, idx, dimension_numbers=_GATHER_DNUMS,
                    slice_sizes=(1,),
                    mode=lax.GatherScatterMode.PROMISE_IN_BOUNDS)


def _splat(val, dtype):
  return jnp.full((16,), val, dtype)


def _worker_id():
  return lax.axis_index("s") * 2 + lax.axis_index("c")


def _nchunks(wid):
  return (NCHUNKS // NW) + jnp.where(wid < (NCHUNKS % NW), 1, 0)


@functools.partial(
    pl.kernel,
    out_type=jax.ShapeDtypeStruct((2 * NR_PAD,), _f32),
    mesh=_MESH,
    scratch_types=[
        pltpu.VMEM((K,), _i32),
        pltpu.VMEM((K,), _i32),
        pltpu.VMEM((K,), _i32),
        pltpu.VMEM((K,), _f32),
        pltpu.VMEM_SHARED((NR_PAD,), _f32),
    ],
)
def _counts_kernel(dst_hbm, et_hbm, zc_hbm,
                   cnt_out,
                   dst_v, typ_v, ci_v, ones_v, acc_sh):
  cid = lax.axis_index("c")
  sid = lax.axis_index("s")
  wid = _worker_id()

  for k in range(K // 16):
    ones_v[pl.ds(k * 16, 16)] = jnp.full((16,), 1.0, _f32)
  seg = NR_PAD // 16
  pltpu.sync_copy(zc_hbm.at[pl.ds(sid * seg, seg)],
                  acc_sh.at[pl.ds(sid * seg, seg)])
  plsc.subcore_barrier()

  def chunk_body(i, carry):
    off = (wid + NW * i) * K
    pltpu.sync_copy(dst_hbm.at[pl.ds(off, K)], dst_v)
    pltpu.sync_copy(et_hbm.at[pl.ds(off, K)], typ_v)
    for k in range(K // 16):
      d16 = dst_v[pl.ds(k * 16, 16)]
      t16 = typ_v[pl.ds(k * 16, 16)]
      ci_v[pl.ds(k * 16, 16)] = d16 * N_REL + t16
    pltpu.sync_copy(ones_v, acc_sh.at[ci_v], add=True)
    return carry

  lax.fori_loop(0, _nchunks(wid), chunk_body, 0)
  plsc.subcore_barrier()
  pltpu.sync_copy(acc_sh.at[pl.ds(sid * seg, seg)],
                  cnt_out.at[pl.ds(cid * NR_PAD + sid * seg, seg)])


@functools.partial(
    pl.kernel,
    out_type=jax.ShapeDtypeStruct((2, N_ACC, D), _f32),
    mesh=_MESH,
    scratch_types=[
        pltpu.VMEM((K,), _i32),
        pltpu.VMEM((K,), _i32),
        pltpu.VMEM((K,), _i32),
        pltpu.VMEM((K,), _i32),
        pltpu.VMEM((K,), _i32),
        pltpu.VMEM((K,), _f32),
        pltpu.VMEM((K, D), _f32),
        pltpu.SemaphoreType.DMA,
        pltpu.VMEM_SHARED((N_ACC, D), _f32),
        pltpu.VMEM_SHARED((NR_PAD,), _f32),
    ],
)
def _agg_kernel(hx_hbm, src_hbm, dst_hbm, et_hbm, recip_hbm, za_hbm,
                out_hbm,
                src_v, dst_v, typ_v, gi_v, ci_v, w_v, rows_v, sem,
                acc_sh, recip_sh):
  cid = lax.axis_index("c")
  sid = lax.axis_index("s")
  wid = _worker_id()

  pltpu.sync_copy(za_hbm.at[pl.ds(sid * RPT, RPT), :],
                  acc_sh.at[pl.ds(sid * RPT, RPT), :])
  seg = NR_PAD // 16
  pltpu.sync_copy(recip_hbm.at[pl.ds(sid * seg, seg)],
                  recip_sh.at[pl.ds(sid * seg, seg)])
  plsc.subcore_barrier()

  def chunk_body(i, carry):
    off = (wid + NW * i) * K
    pltpu.sync_copy(src_hbm.at[pl.ds(off, K)], src_v)
    pltpu.sync_copy(dst_hbm.at[pl.ds(off, K)], dst_v)
    pltpu.sync_copy(et_hbm.at[pl.ds(off, K)], typ_v)
    for k in range(K // 16):
      s16 = src_v[pl.ds(k * 16, 16)]
      d16 = dst_v[pl.ds(k * 16, 16)]
      t16 = typ_v[pl.ds(k * 16, 16)]
      gi_v[pl.ds(k * 16, 16)] = t16 * N_NODES + s16
      ci_v[pl.ds(k * 16, 16)] = d16 * N_REL + t16
    rows_cp = pltpu.make_async_copy(hx_hbm.at[gi_v], rows_v, sem)
    rows_cp.start()
    pltpu.sync_copy(recip_sh.at[ci_v], w_v)
    rows_cp.wait()

    def scale_group(g, c2):
      w16 = w_v[pl.ds(g * 16, 16)]
      for j in range(16):
        wb = _bcast_lane(w16, j)
        e = g * 16 + j
        for k in range(D // 16):
          rows_v[e, pl.ds(k * 16, 16)] = rows_v[e, pl.ds(k * 16, 16)] * wb
      return c2

    lax.fori_loop(0, K // 16, scale_group, 0)
    pltpu.sync_copy(rows_v, acc_sh.at[dst_v], add=True)
    return carry

  lax.fori_loop(0, _nchunks(wid), chunk_body, 0)
  plsc.subcore_barrier()
  pltpu.sync_copy(acc_sh.at[pl.ds(sid * RPT, RPT), :],
                  out_hbm.at[cid, pl.ds(sid * RPT, RPT), :])


def _combine_body(comp_ref, basis_ref, out_ref):
  out_ref[...] = jnp.dot(comp_ref[...], basis_ref[...],
                         preferred_element_type=_f32)


def _combine(comp, basis_flat):
  nb = comp.shape[1]
  f = basis_flat.shape[1]
  blk = 2048
  return pl.pallas_call(
      _combine_body,
      grid=(f // blk,),
      in_specs=[
          pl.BlockSpec((N_REL, nb), lambda i: (0, 0)),
          pl.BlockSpec((nb, blk), lambda i: (0, i)),
      ],
      out_specs=pl.BlockSpec((N_REL, blk), lambda i: (0, i)),
      out_shape=jax.ShapeDtypeStruct((N_REL, f), _f32),
  )(comp, basis_flat)


def _hx_body(x_ref, w_ref, out_ref):
  out_ref[0] = jnp.dot(x_ref[...], w_ref[0], preferred_element_type=_f32)


def _hx(x, w_full):
  bn = 1000
  din = x.shape[1]
  return pl.pallas_call(
      _hx_body,
      grid=(N_NODES // bn, 17),
      in_specs=[
          pl.BlockSpec((bn, din), lambda nb, r: (nb, 0)),
          pl.BlockSpec((1, din, D), lambda nb, r: (r, 0, 0)),
      ],
      out_specs=pl.BlockSpec((1, bn, D), lambda nb, r: (r, nb, 0)),
      out_shape=jax.ShapeDtypeStruct((17, N_NODES, D), _f32),
  )(x, w_full)


def _recip_body(cnt_ref, out_ref):
  c = cnt_ref[0] + cnt_ref[1]
  out_ref[...] = 1.0 / jnp.maximum(c, 1.0)


def _recip(cnt):
  rows = NR_PAD // D
  blk = 128
  out = pl.pallas_call(
      _recip_body,
      grid=(rows // blk,),
      in_specs=[pl.BlockSpec((2, blk, D), lambda i: (0, i, 0))],
      out_specs=pl.BlockSpec((blk, D), lambda i: (i, 0)),
      out_shape=jax.ShapeDtypeStruct((rows, D), _f32),
  )(cnt.reshape(2, rows, D))
  return out.reshape(NR_PAD)


def _z_body(agg_ref, root_ref, bias_ref, out_ref):
  s = agg_ref[0] + agg_ref[1] + root_ref[0] + bias_ref[...]
  out_ref[...] = jnp.where(s >= 0, s, 0.01 * s)


def _z_layer(agg, hx1, bias1):
  bn = 1000
  return pl.pallas_call(
      _z_body,
      grid=(N_NODES // bn,),
      in_specs=[
          pl.BlockSpec((2, bn, D), lambda i: (0, i, 0)),
          pl.BlockSpec((1, bn, D), lambda i: (16, i, 0)),
          pl.BlockSpec((1, D), lambda i: (0, 0)),
      ],
      out_specs=pl.BlockSpec((bn, D), lambda i: (i, 0)),
      out_shape=jax.ShapeDtypeStruct((N_NODES, D), _f32),
  )(agg, hx1, bias1.reshape(1, D))


def _final_body(agg_ref, root_ref, bias_ref, mu_ref, ls_ref):
  s = agg_ref[0] + agg_ref[1] + root_ref[0] + bias_ref[...]
  mu_ref[...] = s[:, :64]
  ls_ref[...] = s[:, 64:]


def _final(agg23, hx23, bias23):
  bn = 1000
  return pl.pallas_call(
      _final_body,
      grid=(N_NODES // bn,),
      in_specs=[
          pl.BlockSpec((2, bn, D), lambda i: (0, i, 0)),
          pl.BlockSpec((1, bn, D), lambda i: (16, i, 0)),
          pl.BlockSpec((1, D), lambda i: (0, 0)),
      ],
      out_specs=[
          pl.BlockSpec((bn, 64), lambda i: (i, 0)),
          pl.BlockSpec((bn, 64), lambda i: (i, 0)),
      ],
      out_shape=(jax.ShapeDtypeStruct((N_NODES, 64), _f32),
                 jax.ShapeDtypeStruct((N_NODES, 64), _f32)),
  )(agg23, hx23, bias23.reshape(1, D))


def kernel(x, edge_index, edge_type, comp1, basis1, root1, bias1,
           comp_mu, basis_mu, root_mu, bias_mu,
           comp_ls, basis_ls, root_ls, bias_ls):
  nb = basis1.shape[0]

  zc = jnp.zeros((NR_PAD,), _f32)
  za = jnp.zeros((N_ACC, D), _f32)
  src_arr = edge_index[0]
  dst_arr = edge_index[1]

  cnt = _counts_kernel(dst_arr, edge_type, zc)
  recip = _recip(cnt)

  w1 = _combine(comp1, basis1.reshape(nb, -1)).reshape(N_REL, D, D)
  w1_full = jnp.concatenate([w1, root1[None]], axis=0)
  hx1 = _hx(x, w1_full)
  agg1 = _agg_kernel(hx1.reshape(17 * N_NODES, D), src_arr, dst_arr,
                     edge_type, recip, za)
  z = _z_layer(agg1, hx1, bias1)

  wmu = _combine(comp_mu, basis_mu.reshape(nb, -1)).reshape(N_REL, D, 64)
  wls = _combine(comp_ls, basis_ls.reshape(nb, -1)).reshape(N_REL, D, 64)
  w23 = jnp.concatenate([wmu, wls], axis=2)
  root23 = jnp.concatenate([root_mu, root_ls], axis=1)
  w23_full = jnp.concatenate([w23, root23[None]], axis=0)
  hx23 = _hx(z, w23_full)
  agg23 = _agg_kernel(hx23.reshape(17 * N_NODES, D), src_arr, dst_arr,
                      edge_type, recip, za)
  bias23 = jnp.concatenate([bias_mu, bias_ls], axis=0)
  mu, logstd = _final(agg23, hx23, bias23)
  return (mu, logstd)


# speedup vs baseline: 14.9041x; 1.1068x over previous
import functools

import jax
import jax.numpy as jnp
from jax import lax
from jax.experimental import pallas as pl
from jax.experimental.pallas import tpu as pltpu
from jax.experimental.pallas import tpu_sc as plsc

N_NODES = 10000
N_EDGES = 320000
N_REL = 16
NR = N_NODES * N_REL
NR_PAD = 163840
D = 128
K = 128
NCHUNKS = N_EDGES // K
NW = 32
N_ACC = 10240
RPT = N_ACC // 16
CPT = 80

_f32 = jnp.float32
_i32 = jnp.int32

_MESH = plsc.VectorSubcoreMesh(
    core_axis_name="c", subcore_axis_name="s", num_cores=2, num_subcores=16)

_GATHER_DNUMS = lax.GatherDimensionNumbers(
    offset_dims=(), collapsed_slice_dims=(0,), start_index_map=(0,))


def _bcast_lane(vec16, j):
  idx = jnp.full((16, 1), j, _i32)
  return lax.gather(vec16, idx, dimension_numbers=_GATHER_DNUMS,
                    slice_sizes=(1,),
                    mode=lax.GatherScatterMode.PROMISE_IN_BOUNDS)


def _splat(val, dtype):
  return jnp.full((16,), val, dtype)


def _worker_id():
  return lax.axis_index("s") * 2 + lax.axis_index("c")


def _nchunks(wid):
  return (NCHUNKS // NW) + jnp.where(wid < (NCHUNKS % NW), 1, 0)


@functools.partial(
    pl.kernel,
    out_type=jax.ShapeDtypeStruct((2 * NR_PAD,), _f32),
    mesh=_MESH,
    scratch_types=[
        pltpu.VMEM((K,), _i32),
        pltpu.VMEM((K,), _i32),
        pltpu.VMEM((K,), _i32),
        pltpu.VMEM((K,), _f32),
        pltpu.VMEM_SHARED((NR_PAD,), _f32),
    ],
)
def _counts_kernel(dst_hbm, et_hbm, zc_hbm,
                   cnt_out,
                   dst_v, typ_v, ci_v, ones_v, acc_sh):
  cid = lax.axis_index("c")
  sid = lax.axis_index("s")
  wid = _worker_id()

  for k in range(K // 16):
    ones_v[pl.ds(k * 16, 16)] = jnp.full((16,), 1.0, _f32)
  seg = NR_PAD // 16
  pltpu.sync_copy(zc_hbm.at[pl.ds(sid * seg, seg)],
                  acc_sh.at[pl.ds(sid * seg, seg)])
  plsc.subcore_barrier()

  def chunk_body(i, carry):
    off = (wid + NW * i) * K
    pltpu.sync_copy(dst_hbm.at[pl.ds(off, K)], dst_v)
    pltpu.sync_copy(et_hbm.at[pl.ds(off, K)], typ_v)
    for k in range(K // 16):
      d16 = dst_v[pl.ds(k * 16, 16)]
      t16 = typ_v[pl.ds(k * 16, 16)]
      ci_v[pl.ds(k * 16, 16)] = d16 * N_REL + t16
    pltpu.sync_copy(ones_v, acc_sh.at[ci_v], add=True)
    return carry

  lax.fori_loop(0, _nchunks(wid), chunk_body, 0)
  plsc.subcore_barrier()
  pltpu.sync_copy(acc_sh.at[pl.ds(sid * seg, seg)],
                  cnt_out.at[pl.ds(cid * NR_PAD + sid * seg, seg)])


@functools.partial(
    pl.kernel,
    out_type=jax.ShapeDtypeStruct((2, N_ACC, D), _f32),
    mesh=_MESH,
    scratch_types=[
        pltpu.VMEM((K,), _i32), pltpu.VMEM((K,), _i32),
        pltpu.VMEM((K,), _i32), pltpu.VMEM((K,), _i32),
        pltpu.VMEM((K,), _i32), pltpu.VMEM((K,), _i32),
        pltpu.VMEM((K,), _i32),
        pltpu.VMEM((K,), _i32),
        pltpu.VMEM((K,), _f32),
        pltpu.VMEM((K, D), _f32),
        pltpu.SemaphoreType.DMA, pltpu.SemaphoreType.DMA,
        pltpu.SemaphoreType.DMA,
        pltpu.VMEM_SHARED((N_ACC, D), _f32),
        pltpu.VMEM_SHARED((NR_PAD,), _f32),
    ],
)
def _agg_kernel(hx_hbm, src_hbm, dst_hbm, et_hbm, recip_hbm, za_hbm,
                out_hbm,
                src_v0, src_v1, dst_v0, dst_v1, typ_v0, typ_v1,
                gi_v, ci_v, w_v, rows_v,
                isem0, isem1, gsem,
                acc_sh, recip_sh):
  cid = lax.axis_index("c")
  sid = lax.axis_index("s")
  wid = _worker_id()

  pltpu.sync_copy(za_hbm.at[pl.ds(sid * RPT, RPT), :],
                  acc_sh.at[pl.ds(sid * RPT, RPT), :])
  seg = NR_PAD // 16
  pltpu.sync_copy(recip_hbm.at[pl.ds(sid * seg, seg)],
                  recip_sh.at[pl.ds(sid * seg, seg)])
  plsc.subcore_barrier()

  def off_of(e):
    c = jnp.minimum(wid + NW * e, NCHUNKS - 1)
    return c * K

  def fire_in(e, sv, dv, tv, sem):
    o = off_of(e)
    pltpu.make_async_copy(src_hbm.at[pl.ds(o, K)], sv, sem).start()
    pltpu.make_async_copy(dst_hbm.at[pl.ds(o, K)], dv, sem).start()
    pltpu.make_async_copy(et_hbm.at[pl.ds(o, K)], tv, sem).start()

  def wait_in(e, sv, dv, tv, sem):
    o = off_of(e)
    pltpu.make_async_copy(src_hbm.at[pl.ds(o, K)], sv, sem).wait()
    pltpu.make_async_copy(dst_hbm.at[pl.ds(o, K)], dv, sem).wait()
    pltpu.make_async_copy(et_hbm.at[pl.ds(o, K)], tv, sem).wait()

  def process(e, sv, dv, tv):
    for k in range(K // 16):
      s16 = sv[pl.ds(k * 16, 16)]
      d16 = dv[pl.ds(k * 16, 16)]
      t16 = tv[pl.ds(k * 16, 16)]
      gi_v[pl.ds(k * 16, 16)] = t16 * N_NODES + s16
      ci_v[pl.ds(k * 16, 16)] = d16 * N_REL + t16
    rows_cp = pltpu.make_async_copy(hx_hbm.at[gi_v], rows_v, gsem)
    rows_cp.start()
    pltpu.sync_copy(recip_sh.at[ci_v], w_v)
    rows_cp.wait()
    c = wid + NW * e
    flag = jnp.where(c < NCHUNKS, 1.0, 0.0).astype(_f32)
    fb = jnp.broadcast_to(flag, (16,))
    for g in range(K // 16):
      w16 = w_v[pl.ds(g * 16, 16)] * fb
      for j in range(16):
        wb = _bcast_lane(w16, j)
        e2 = g * 16 + j
        for k in range(D // 16):
          rows_v[e2, pl.ds(k * 16, 16)] = rows_v[e2, pl.ds(k * 16, 16)] * wb
    pltpu.sync_copy(rows_v, acc_sh.at[dv], add=True)

  fire_in(jnp.int32(0), src_v0, dst_v0, typ_v0, isem0)

  def pair(i, carry):
    e0 = 2 * i
    wait_in(e0, src_v0, dst_v0, typ_v0, isem0)
    fire_in(e0 + 1, src_v1, dst_v1, typ_v1, isem1)
    process(e0, src_v0, dst_v0, typ_v0)
    e1 = e0 + 1
    wait_in(e1, src_v1, dst_v1, typ_v1, isem1)
    fire_in(e1 + 1, src_v0, dst_v0, typ_v0, isem0)
    process(e1, src_v1, dst_v1, typ_v1)
    return carry

  lax.fori_loop(0, CPT // 2, pair, 0)
  wait_in(jnp.int32(CPT), src_v0, dst_v0, typ_v0, isem0)
  plsc.subcore_barrier()
  pltpu.sync_copy(acc_sh.at[pl.ds(sid * RPT, RPT), :],
                  out_hbm.at[cid, pl.ds(sid * RPT, RPT), :])


def _combine_body(comp_ref, basis_ref, out_ref):
  out_ref[...] = jnp.dot(comp_ref[...], basis_ref[...],
                         preferred_element_type=_f32)


def _combine(comp, basis_flat):
  nb = comp.shape[1]
  f = basis_flat.shape[1]
  blk = 2048
  return pl.pallas_call(
      _combine_body,
      grid=(f // blk,),
      in_specs=[
          pl.BlockSpec((N_REL, nb), lambda i: (0, 0)),
          pl.BlockSpec((nb, blk), lambda i: (0, i)),
      ],
      out_specs=pl.BlockSpec((N_REL, blk), lambda i: (0, i)),
      out_shape=jax.ShapeDtypeStruct((N_REL, f), _f32),
  )(comp, basis_flat)


def _hx_body(x_ref, w_ref, out_ref):
  out_ref[0] = jnp.dot(x_ref[...], w_ref[0], preferred_element_type=_f32)


def _hx(x, w_full):
  bn = 1000
  din = x.shape[1]
  return pl.pallas_call(
      _hx_body,
      grid=(N_NODES // bn, 17),
      in_specs=[
          pl.BlockSpec((bn, din), lambda nb, r: (nb, 0)),
          pl.BlockSpec((1, din, D), lambda nb, r: (r, 0, 0)),
      ],
      out_specs=pl.BlockSpec((1, bn, D), lambda nb, r: (r, nb, 0)),
      out_shape=jax.ShapeDtypeStruct((17, N_NODES, D), _f32),
  )(x, w_full)


def _recip_body(cnt_ref, out_ref):
  c = cnt_ref[0] + cnt_ref[1]
  out_ref[...] = 1.0 / jnp.maximum(c, 1.0)


def _recip(cnt):
  rows = NR_PAD // D
  blk = 128
  out = pl.pallas_call(
      _recip_body,
      grid=(rows // blk,),
      in_specs=[pl.BlockSpec((2, blk, D), lambda i: (0, i, 0))],
      out_specs=pl.BlockSpec((blk, D), lambda i: (i, 0)),
      out_shape=jax.ShapeDtypeStruct((rows, D), _f32),
  )(cnt.reshape(2, rows, D))
  return out.reshape(NR_PAD)


def _z_body(agg_ref, root_ref, bias_ref, out_ref):
  s = agg_ref[0] + agg_ref[1] + root_ref[0] + bias_ref[...]
  out_ref[...] = jnp.where(s >= 0, s, 0.01 * s)


def _z_layer(agg, hx1, bias1):
  bn = 1000
  return pl.pallas_call(
      _z_body,
      grid=(N_NODES // bn,),
      in_specs=[
          pl.BlockSpec((2, bn, D), lambda i: (0, i, 0)),
          pl.BlockSpec((1, bn, D), lambda i: (16, i, 0)),
          pl.BlockSpec((1, D), lambda i: (0, 0)),
      ],
      out_specs=pl.BlockSpec((bn, D), lambda i: (i, 0)),
      out_shape=jax.ShapeDtypeStruct((N_NODES, D), _f32),
  )(agg, hx1, bias1.reshape(1, D))


def _final_body(agg_ref, root_ref, bias_ref, mu_ref, ls_ref):
  s = agg_ref[0] + agg_ref[1] + root_ref[0] + bias_ref[...]
  mu_ref[...] = s[:, :64]
  ls_ref[...] = s[:, 64:]


def _final(agg23, hx23, bias23):
  bn = 1000
  return pl.pallas_call(
      _final_body,
      grid=(N_NODES // bn,),
      in_specs=[
          pl.BlockSpec((2, bn, D), lambda i: (0, i, 0)),
          pl.BlockSpec((1, bn, D), lambda i: (16, i, 0)),
          pl.BlockSpec((1, D), lambda i: (0, 0)),
      ],
      out_specs=[
          pl.BlockSpec((bn, 64), lambda i: (i, 0)),
          pl.BlockSpec((bn, 64), lambda i: (i, 0)),
      ],
      out_shape=(jax.ShapeDtypeStruct((N_NODES, 64), _f32),
                 jax.ShapeDtypeStruct((N_NODES, 64), _f32)),
  )(agg23, hx23, bias23.reshape(1, D))


def kernel(x, edge_index, edge_type, comp1, basis1, root1, bias1,
           comp_mu, basis_mu, root_mu, bias_mu,
           comp_ls, basis_ls, root_ls, bias_ls):
  nb = basis1.shape[0]

  zc = jnp.zeros((NR_PAD,), _f32)
  za = jnp.zeros((N_ACC, D), _f32)
  src_arr = edge_index[0]
  dst_arr = edge_index[1]

  cnt = _counts_kernel(dst_arr, edge_type, zc)
  recip = _recip(cnt)

  w1 = _combine(comp1, basis1.reshape(nb, -1)).reshape(N_REL, D, D)
  w1_full = jnp.concatenate([w1, root1[None]], axis=0)
  hx1 = _hx(x, w1_full)
  agg1 = _agg_kernel(hx1.reshape(17 * N_NODES, D), src_arr, dst_arr,
                     edge_type, recip, za)
  z = _z_layer(agg1, hx1, bias1)

  wmu = _combine(comp_mu, basis_mu.reshape(nb, -1)).reshape(N_REL, D, 64)
  wls = _combine(comp_ls, basis_ls.reshape(nb, -1)).reshape(N_REL, D, 64)
  w23 = jnp.concatenate([wmu, wls], axis=2)
  root23 = jnp.concatenate([root_mu, root_ls], axis=1)
  w23_full = jnp.concatenate([w23, root23[None]], axis=0)
  hx23 = _hx(z, w23_full)
  agg23 = _agg_kernel(hx23.reshape(17 * N_NODES, D), src_arr, dst_arr,
                      edge_type, recip, za)
  bias23 = jnp.concatenate([bias_mu, bias_ls], axis=0)
  mu, logstd = _final(agg23, hx23, bias23)
  return (mu, logstd)


# speedup vs baseline: 17.6330x; 1.1831x over previous
import functools

import jax
import jax.numpy as jnp
from jax import lax
from jax.experimental import pallas as pl
from jax.experimental.pallas import tpu as pltpu
from jax.experimental.pallas import tpu_sc as plsc

N_NODES = 10000
N_EDGES = 320000
N_REL = 16
NR = N_NODES * N_REL
NR_PAD = 163840
D = 128
K = 128
NCHUNKS = N_EDGES // K
NW = 32
N_ACC = 10240
RPT = N_ACC // 16
CPT = 80

_f32 = jnp.float32
_i32 = jnp.int32

_MESH = plsc.VectorSubcoreMesh(
    core_axis_name="c", subcore_axis_name="s", num_cores=2, num_subcores=16)

_GATHER_DNUMS = lax.GatherDimensionNumbers(
    offset_dims=(), collapsed_slice_dims=(0,), start_index_map=(0,))


def _bcast_lane(vec16, j):
  idx = jnp.full((16, 1), j, _i32)
  return lax.gather(vec16, idx, dimension_numbers=_GATHER_DNUMS,
                    slice_sizes=(1,),
                    mode=lax.GatherScatterMode.PROMISE_IN_BOUNDS)


def _splat(val, dtype):
  return jnp.full((16,), val, dtype)


def _worker_id():
  return lax.axis_index("s") * 2 + lax.axis_index("c")


def _nchunks(wid):
  return (NCHUNKS // NW) + jnp.where(wid < (NCHUNKS % NW), 1, 0)


@functools.partial(
    pl.kernel,
    out_type=jax.ShapeDtypeStruct((2 * NR_PAD,), _f32),
    mesh=_MESH,
    scratch_types=[
        pltpu.VMEM((K,), _i32),
        pltpu.VMEM((K,), _i32),
        pltpu.VMEM((K,), _i32),
        pltpu.VMEM((K,), _f32),
        pltpu.VMEM_SHARED((NR_PAD,), _f32),
    ],
)
def _counts_kernel(dst_hbm, et_hbm, zc_hbm,
                   cnt_out,
                   dst_v, typ_v, ci_v, ones_v, acc_sh):
  cid = lax.axis_index("c")
  sid = lax.axis_index("s")
  wid = _worker_id()

  for k in range(K // 16):
    ones_v[pl.ds(k * 16, 16)] = jnp.full((16,), 1.0, _f32)
  seg = NR_PAD // 16
  pltpu.sync_copy(zc_hbm.at[pl.ds(sid * seg, seg)],
                  acc_sh.at[pl.ds(sid * seg, seg)])
  plsc.subcore_barrier()

  def chunk_body(i, carry):
    off = (wid + NW * i) * K
    pltpu.sync_copy(dst_hbm.at[pl.ds(off, K)], dst_v)
    pltpu.sync_copy(et_hbm.at[pl.ds(off, K)], typ_v)
    for k in range(K // 16):
      d16 = dst_v[pl.ds(k * 16, 16)]
      t16 = typ_v[pl.ds(k * 16, 16)]
      ci_v[pl.ds(k * 16, 16)] = d16 * N_REL + t16
    pltpu.sync_copy(ones_v, acc_sh.at[ci_v], add=True)
    return carry

  lax.fori_loop(0, _nchunks(wid), chunk_body, 0)
  plsc.subcore_barrier()
  pltpu.sync_copy(acc_sh.at[pl.ds(sid * seg, seg)],
                  cnt_out.at[pl.ds(cid * NR_PAD + sid * seg, seg)])


@functools.partial(
    pl.kernel,
    out_type=jax.ShapeDtypeStruct((2, N_ACC, D), _f32),
    mesh=_MESH,
    scratch_types=[
        pltpu.VMEM((K,), _i32), pltpu.VMEM((K,), _i32),
        pltpu.VMEM((K,), _i32), pltpu.VMEM((K,), _i32),
        pltpu.VMEM((K,), _i32), pltpu.VMEM((K,), _i32),
        pltpu.VMEM((K,), _i32), pltpu.VMEM((K,), _i32),
        pltpu.VMEM((K,), _i32), pltpu.VMEM((K,), _i32),
        pltpu.VMEM((K,), _i32), pltpu.VMEM((K,), _i32),
        pltpu.VMEM((K,), _f32), pltpu.VMEM((K,), _f32),
        pltpu.VMEM((K, D), _f32), pltpu.VMEM((K, D), _f32),
        pltpu.SemaphoreType.DMA, pltpu.SemaphoreType.DMA,
        pltpu.SemaphoreType.DMA, pltpu.SemaphoreType.DMA,
        pltpu.VMEM_SHARED((N_ACC, D), _f32),
        pltpu.VMEM_SHARED((NR_PAD,), _f32),
    ],
)
def _agg_kernel(hx_hbm, src_hbm, dst_hbm, et_hbm, recip_hbm, za_hbm,
                out_hbm,
                src_v0, src_v1, dst_v0, dst_v1, typ_v0, typ_v1,
                gi_v0, gi_v1, ci_v0, ci_v1, di_v0, di_v1,
                w_v0, w_v1, rows_v0, rows_v1,
                isem0, isem1, gsem0, gsem1,
                acc_sh, recip_sh):
  cid = lax.axis_index("c")
  sid = lax.axis_index("s")
  wid = _worker_id()

  pltpu.sync_copy(za_hbm.at[pl.ds(sid * RPT, RPT), :],
                  acc_sh.at[pl.ds(sid * RPT, RPT), :])
  seg = NR_PAD // 16
  pltpu.sync_copy(recip_hbm.at[pl.ds(sid * seg, seg)],
                  recip_sh.at[pl.ds(sid * seg, seg)])
  plsc.subcore_barrier()

  def off_of(e):
    c = jnp.minimum(wid + NW * e, NCHUNKS - 1)
    return c * K

  def fire_in(e, sv, dv, tv, sem):
    o = off_of(e)
    pltpu.make_async_copy(src_hbm.at[pl.ds(o, K)], sv, sem).start()
    pltpu.make_async_copy(dst_hbm.at[pl.ds(o, K)], dv, sem).start()
    pltpu.make_async_copy(et_hbm.at[pl.ds(o, K)], tv, sem).start()

  def wait_in(e, sv, dv, tv, sem):
    o = off_of(e)
    pltpu.make_async_copy(src_hbm.at[pl.ds(o, K)], sv, sem).wait()
    pltpu.make_async_copy(dst_hbm.at[pl.ds(o, K)], dv, sem).wait()
    pltpu.make_async_copy(et_hbm.at[pl.ds(o, K)], tv, sem).wait()

  def front(e, sv, dv, tv, gv, cv, div, wv, rv, isem, gsem, last):
    wait_in(e, sv, dv, tv, isem)
    for k in range(K // 16):
      s16 = sv[pl.ds(k * 16, 16)]
      d16 = dv[pl.ds(k * 16, 16)]
      t16 = tv[pl.ds(k * 16, 16)]
      gv[pl.ds(k * 16, 16)] = t16 * N_NODES + s16
      cv[pl.ds(k * 16, 16)] = d16 * N_REL + t16
      div[pl.ds(k * 16, 16)] = d16
    pltpu.make_async_copy(hx_hbm.at[gv], rv, gsem).start()
    pltpu.sync_copy(recip_sh.at[cv], wv)
    if not last:
      fire_in(e + 2, sv, dv, tv, isem)

  def back(e, gv, wv, rv, div, gsem):
    pltpu.make_async_copy(hx_hbm.at[gv], rv, gsem).wait()
    c = wid + NW * e
    flag = jnp.where(c < NCHUNKS, 1.0, 0.0).astype(_f32)
    fb = jnp.broadcast_to(flag, (16,))
    for g in range(K // 16):
      w16 = wv[pl.ds(g * 16, 16)] * fb
      for j in range(16):
        wb = _bcast_lane(w16, j)
        e2 = g * 16 + j
        for k in range(D // 16):
          rv[e2, pl.ds(k * 16, 16)] = rv[e2, pl.ds(k * 16, 16)] * wb
    pltpu.sync_copy(rv, acc_sh.at[div], add=True)

  fire_in(jnp.int32(0), src_v0, dst_v0, typ_v0, isem0)
  fire_in(jnp.int32(1), src_v1, dst_v1, typ_v1, isem1)
  front(jnp.int32(0), src_v0, dst_v0, typ_v0, gi_v0, ci_v0, di_v0,
        w_v0, rows_v0, isem0, gsem0, False)

  def pair(i, carry):
    e1 = 2 * i + 1
    front(e1, src_v1, dst_v1, typ_v1, gi_v1, ci_v1, di_v1,
          w_v1, rows_v1, isem1, gsem1, False)
    back(e1 - 1, gi_v0, w_v0, rows_v0, di_v0, gsem0)
    e2 = e1 + 1
    front(e2, src_v0, dst_v0, typ_v0, gi_v0, ci_v0, di_v0,
          w_v0, rows_v0, isem0, gsem0, False)
    back(e2 - 1, gi_v1, w_v1, rows_v1, di_v1, gsem1)
    return carry

  lax.fori_loop(0, (CPT - 2) // 2, pair, 0)
  front(jnp.int32(CPT - 1), src_v1, dst_v1, typ_v1, gi_v1, ci_v1, di_v1,
        w_v1, rows_v1, isem1, gsem1, True)
  back(jnp.int32(CPT - 2), gi_v0, w_v0, rows_v0, di_v0, gsem0)
  back(jnp.int32(CPT - 1), gi_v1, w_v1, rows_v1, di_v1, gsem1)
  wait_in(jnp.int32(CPT), src_v0, dst_v0, typ_v0, isem0)
  plsc.subcore_barrier()
  pltpu.sync_copy(acc_sh.at[pl.ds(sid * RPT, RPT), :],
                  out_hbm.at[cid, pl.ds(sid * RPT, RPT), :])


def _combine_body(comp_ref, basis_ref, out_ref):
  out_ref[...] = jnp.dot(comp_ref[...], basis_ref[...],
                         preferred_element_type=_f32)


def _combine(comp, basis_flat):
  nb = comp.shape[1]
  f = basis_flat.shape[1]
  blk = 2048
  return pl.pallas_call(
      _combine_body,
      grid=(f // blk,),
      in_specs=[
          pl.BlockSpec((N_REL, nb), lambda i: (0, 0)),
          pl.BlockSpec((nb, blk), lambda i: (0, i)),
      ],
      out_specs=pl.BlockSpec((N_REL, blk), lambda i: (0, i)),
      out_shape=jax.ShapeDtypeStruct((N_REL, f), _f32),
  )(comp, basis_flat)


def _hx_body(x_ref, w_ref, out_ref):
  out_ref[0] = jnp.dot(x_ref[...], w_ref[0], preferred_element_type=_f32)


def _hx(x, w_full):
  bn = 1000
  din = x.shape[1]
  return pl.pallas_call(
      _hx_body,
      grid=(N_NODES // bn, 17),
      in_specs=[
          pl.BlockSpec((bn, din), lambda nb, r: (nb, 0)),
          pl.BlockSpec((1, din, D), lambda nb, r: (r, 0, 0)),
      ],
      out_specs=pl.BlockSpec((1, bn, D), lambda nb, r: (r, nb, 0)),
      out_shape=jax.ShapeDtypeStruct((17, N_NODES, D), _f32),
  )(x, w_full)


def _recip_body(cnt_ref, out_ref):
  c = cnt_ref[0] + cnt_ref[1]
  out_ref[...] = 1.0 / jnp.maximum(c, 1.0)


def _recip(cnt):
  rows = NR_PAD // D
  blk = 128
  out = pl.pallas_call(
      _recip_body,
      grid=(rows // blk,),
      in_specs=[pl.BlockSpec((2, blk, D), lambda i: (0, i, 0))],
      out_specs=pl.BlockSpec((blk, D), lambda i: (i, 0)),
      out_shape=jax.ShapeDtypeStruct((rows, D), _f32),
  )(cnt.reshape(2, rows, D))
  return out.reshape(NR_PAD)


def _z_body(agg_ref, root_ref, bias_ref, out_ref):
  s = agg_ref[0] + agg_ref[1] + root_ref[0] + bias_ref[...]
  out_ref[...] = jnp.where(s >= 0, s, 0.01 * s)


def _z_layer(agg, hx1, bias1):
  bn = 1000
  return pl.pallas_call(
      _z_body,
      grid=(N_NODES // bn,),
      in_specs=[
          pl.BlockSpec((2, bn, D), lambda i: (0, i, 0)),
          pl.BlockSpec((1, bn, D), lambda i: (16, i, 0)),
          pl.BlockSpec((1, D), lambda i: (0, 0)),
      ],
      out_specs=pl.BlockSpec((bn, D), lambda i: (i, 0)),
      out_shape=jax.ShapeDtypeStruct((N_NODES, D), _f32),
  )(agg, hx1, bias1.reshape(1, D))


def _final_body(agg_ref, root_ref, bias_ref, mu_ref, ls_ref):
  s = agg_ref[0] + agg_ref[1] + root_ref[0] + bias_ref[...]
  mu_ref[...] = s[:, :64]
  ls_ref[...] = s[:, 64:]


def _final(agg23, hx23, bias23):
  bn = 1000
  return pl.pallas_call(
      _final_body,
      grid=(N_NODES // bn,),
      in_specs=[
          pl.BlockSpec((2, bn, D), lambda i: (0, i, 0)),
          pl.BlockSpec((1, bn, D), lambda i: (16, i, 0)),
          pl.BlockSpec((1, D), lambda i: (0, 0)),
      ],
      out_specs=[
          pl.BlockSpec((bn, 64), lambda i: (i, 0)),
          pl.BlockSpec((bn, 64), lambda i: (i, 0)),
      ],
      out_shape=(jax.ShapeDtypeStruct((N_NODES, 64), _f32),
                 jax.ShapeDtypeStruct((N_NODES, 64), _f32)),
  )(agg23, hx23, bias23.reshape(1, D))


def kernel(x, edge_index, edge_type, comp1, basis1, root1, bias1,
           comp_mu, basis_mu, root_mu, bias_mu,
           comp_ls, basis_ls, root_ls, bias_ls):
  nb = basis1.shape[0]

  zc = jnp.zeros((NR_PAD,), _f32)
  za = jnp.zeros((N_ACC, D), _f32)
  src_arr = edge_index[0]
  dst_arr = edge_index[1]

  cnt = _counts_kernel(dst_arr, edge_type, zc)
  recip = _recip(cnt)

  w1 = _combine(comp1, basis1.reshape(nb, -1)).reshape(N_REL, D, D)
  w1_full = jnp.concatenate([w1, root1[None]], axis=0)
  hx1 = _hx(x, w1_full)
  agg1 = _agg_kernel(hx1.reshape(17 * N_NODES, D), src_arr, dst_arr,
                     edge_type, recip, za)
  z = _z_layer(agg1, hx1, bias1)

  wmu = _combine(comp_mu, basis_mu.reshape(nb, -1)).reshape(N_REL, D, 64)
  wls = _combine(comp_ls, basis_ls.reshape(nb, -1)).reshape(N_REL, D, 64)
  w23 = jnp.concatenate([wmu, wls], axis=2)
  root23 = jnp.concatenate([root_mu, root_ls], axis=1)
  w23_full = jnp.concatenate([w23, root23[None]], axis=0)
  hx23 = _hx(z, w23_full)
  agg23 = _agg_kernel(hx23.reshape(17 * N_NODES, D), src_arr, dst_arr,
                      edge_type, recip, za)
  bias23 = jnp.concatenate([bias_mu, bias_ls], axis=0)
  mu, logstd = _final(agg23, hx23, bias23)
  return (mu, logstd)


# speedup vs baseline: 18.1803x; 1.0310x over previous
import functools

import jax
import jax.numpy as jnp
from jax import lax
from jax.experimental import pallas as pl
from jax.experimental.pallas import tpu as pltpu
from jax.experimental.pallas import tpu_sc as plsc

N_NODES = 10000
N_EDGES = 320000
N_REL = 16
NR = N_NODES * N_REL
NR_PAD = 163840
D = 128
K = 128
NCHUNKS = N_EDGES // K
NW = 32
N_ACC = 10240
RPT = N_ACC // 16
CPT = 80

_f32 = jnp.float32
_i32 = jnp.int32

_MESH = plsc.VectorSubcoreMesh(
    core_axis_name="c", subcore_axis_name="s", num_cores=2, num_subcores=16)

_GATHER_DNUMS = lax.GatherDimensionNumbers(
    offset_dims=(), collapsed_slice_dims=(0,), start_index_map=(0,))


def _bcast_lane(vec16, j):
  idx = jnp.full((16, 1), j, _i32)
  return lax.gather(vec16, idx, dimension_numbers=_GATHER_DNUMS,
                    slice_sizes=(1,),
                    mode=lax.GatherScatterMode.PROMISE_IN_BOUNDS)


def _splat(val, dtype):
  return jnp.full((16,), val, dtype)


def _worker_id():
  return lax.axis_index("s") * 2 + lax.axis_index("c")


def _nchunks(wid):
  return (NCHUNKS // NW) + jnp.where(wid < (NCHUNKS % NW), 1, 0)


@functools.partial(
    pl.kernel,
    out_type=jax.ShapeDtypeStruct((2 * NR_PAD,), _f32),
    mesh=_MESH,
    scratch_types=[
        pltpu.VMEM((K,), _i32), pltpu.VMEM((K,), _i32),
        pltpu.VMEM((K,), _i32), pltpu.VMEM((K,), _i32),
        pltpu.VMEM((K,), _i32),
        pltpu.VMEM((K,), _f32),
        pltpu.SemaphoreType.DMA, pltpu.SemaphoreType.DMA,
        pltpu.VMEM_SHARED((NR_PAD,), _f32),
    ],
)
def _counts_kernel(dst_hbm, et_hbm, zc_hbm,
                   cnt_out,
                   dst_v0, dst_v1, typ_v0, typ_v1, ci_v, ones_v,
                   isem0, isem1, acc_sh):
  cid = lax.axis_index("c")
  sid = lax.axis_index("s")
  wid = _worker_id()

  seg = NR_PAD // 16
  pltpu.sync_copy(zc_hbm.at[pl.ds(sid * seg, seg)],
                  acc_sh.at[pl.ds(sid * seg, seg)])
  plsc.subcore_barrier()

  def off_of(e):
    c = jnp.minimum(wid + NW * e, NCHUNKS - 1)
    return c * K

  def fire_in(e, dv, tv, sem):
    o = off_of(e)
    pltpu.make_async_copy(dst_hbm.at[pl.ds(o, K)], dv, sem).start()
    pltpu.make_async_copy(et_hbm.at[pl.ds(o, K)], tv, sem).start()

  def wait_in(e, dv, tv, sem):
    o = off_of(e)
    pltpu.make_async_copy(dst_hbm.at[pl.ds(o, K)], dv, sem).wait()
    pltpu.make_async_copy(et_hbm.at[pl.ds(o, K)], tv, sem).wait()

  def process(e, dv, tv):
    c = wid + NW * e
    flag = jnp.where(c < NCHUNKS, 1.0, 0.0).astype(_f32)
    fb = jnp.broadcast_to(flag, (16,))
    for k in range(K // 16):
      d16 = dv[pl.ds(k * 16, 16)]
      t16 = tv[pl.ds(k * 16, 16)]
      ci_v[pl.ds(k * 16, 16)] = d16 * N_REL + t16
      ones_v[pl.ds(k * 16, 16)] = fb
    pltpu.sync_copy(ones_v, acc_sh.at[ci_v], add=True)

  fire_in(jnp.int32(0), dst_v0, typ_v0, isem0)

  def pair(i, carry):
    e0 = 2 * i
    wait_in(e0, dst_v0, typ_v0, isem0)
    fire_in(e0 + 1, dst_v1, typ_v1, isem1)
    process(e0, dst_v0, typ_v0)
    e1 = e0 + 1
    wait_in(e1, dst_v1, typ_v1, isem1)
    fire_in(e1 + 1, dst_v0, typ_v0, isem0)
    process(e1, dst_v1, typ_v1)
    return carry

  lax.fori_loop(0, CPT // 2, pair, 0)
  wait_in(jnp.int32(CPT), dst_v0, typ_v0, isem0)
  plsc.subcore_barrier()
  pltpu.sync_copy(acc_sh.at[pl.ds(sid * seg, seg)],
                  cnt_out.at[pl.ds(cid * NR_PAD + sid * seg, seg)])


@functools.partial(
    pl.kernel,
    out_type=jax.ShapeDtypeStruct((2, N_ACC, D), _f32),
    mesh=_MESH,
    scratch_types=[
        pltpu.VMEM((K,), _i32), pltpu.VMEM((K,), _i32),
        pltpu.VMEM((K,), _i32), pltpu.VMEM((K,), _i32),
        pltpu.VMEM((K,), _i32), pltpu.VMEM((K,), _i32),
        pltpu.VMEM((K,), _i32), pltpu.VMEM((K,), _i32),
        pltpu.VMEM((K,), _i32), pltpu.VMEM((K,), _i32),
        pltpu.VMEM((K,), _i32), pltpu.VMEM((K,), _i32),
        pltpu.VMEM((K,), _f32), pltpu.VMEM((K,), _f32),
        pltpu.VMEM((K, D), _f32), pltpu.VMEM((K, D), _f32),
        pltpu.SemaphoreType.DMA, pltpu.SemaphoreType.DMA,
        pltpu.SemaphoreType.DMA, pltpu.SemaphoreType.DMA,
        pltpu.VMEM_SHARED((N_ACC, D), _f32),
    ],
)
def _agg_kernel(hx_hbm, src_hbm, dst_hbm, et_hbm, recip_hbm, za_hbm,
                out_hbm,
                src_v0, src_v1, dst_v0, dst_v1, typ_v0, typ_v1,
                gi_v0, gi_v1, ci_v0, ci_v1, di_v0, di_v1,
                w_v0, w_v1, rows_v0, rows_v1,
                isem0, isem1, gsem0, gsem1,
                acc_sh):
  cid = lax.axis_index("c")
  sid = lax.axis_index("s")
  wid = _worker_id()

  pltpu.sync_copy(za_hbm.at[pl.ds(sid * RPT, RPT), :],
                  acc_sh.at[pl.ds(sid * RPT, RPT), :])
  plsc.subcore_barrier()

  def off_of(e):
    c = jnp.minimum(wid + NW * e, NCHUNKS - 1)
    return c * K

  def fire_in(e, sv, dv, tv, sem):
    o = off_of(e)
    pltpu.make_async_copy(src_hbm.at[pl.ds(o, K)], sv, sem).start()
    pltpu.make_async_copy(dst_hbm.at[pl.ds(o, K)], dv, sem).start()
    pltpu.make_async_copy(et_hbm.at[pl.ds(o, K)], tv, sem).start()

  def wait_in(e, sv, dv, tv, sem):
    o = off_of(e)
    pltpu.make_async_copy(src_hbm.at[pl.ds(o, K)], sv, sem).wait()
    pltpu.make_async_copy(dst_hbm.at[pl.ds(o, K)], dv, sem).wait()
    pltpu.make_async_copy(et_hbm.at[pl.ds(o, K)], tv, sem).wait()

  def front(e, sv, dv, tv, gv, cv, div, wv, rv, isem, gsem, last):
    wait_in(e, sv, dv, tv, isem)
    for k in range(K // 16):
      s16 = sv[pl.ds(k * 16, 16)]
      d16 = dv[pl.ds(k * 16, 16)]
      t16 = tv[pl.ds(k * 16, 16)]
      gv[pl.ds(k * 16, 16)] = t16 * N_NODES + s16
      cv[pl.ds(k * 16, 16)] = d16 * N_REL + t16
      div[pl.ds(k * 16, 16)] = d16
    pltpu.make_async_copy(hx_hbm.at[gv], rv, gsem).start()
    pltpu.make_async_copy(recip_hbm.at[cv], wv, gsem).start()
    if not last:
      fire_in(e + 2, sv, dv, tv, isem)

  def back(e, gv, cv, wv, rv, div, gsem):
    pltpu.make_async_copy(hx_hbm.at[gv], rv, gsem).wait()
    pltpu.make_async_copy(recip_hbm.at[cv], wv, gsem).wait()
    c = wid + NW * e
    flag = jnp.where(c < NCHUNKS, 1.0, 0.0).astype(_f32)
    fb = jnp.broadcast_to(flag, (16,))
    for g in range(K // 16):
      w16 = wv[pl.ds(g * 16, 16)] * fb
      for j in range(16):
        wb = _bcast_lane(w16, j)
        e2 = g * 16 + j
        for k in range(D // 16):
          rv[e2, pl.ds(k * 16, 16)] = rv[e2, pl.ds(k * 16, 16)] * wb
    pltpu.sync_copy(rv, acc_sh.at[div], add=True)

  fire_in(jnp.int32(0), src_v0, dst_v0, typ_v0, isem0)
  fire_in(jnp.int32(1), src_v1, dst_v1, typ_v1, isem1)
  front(jnp.int32(0), src_v0, dst_v0, typ_v0, gi_v0, ci_v0, di_v0,
        w_v0, rows_v0, isem0, gsem0, False)

  def pair(i, carry):
    e1 = 2 * i + 1
    front(e1, src_v1, dst_v1, typ_v1, gi_v1, ci_v1, di_v1,
          w_v1, rows_v1, isem1, gsem1, False)
    back(e1 - 1, gi_v0, ci_v0, w_v0, rows_v0, di_v0, gsem0)
    e2 = e1 + 1
    front(e2, src_v0, dst_v0, typ_v0, gi_v0, ci_v0, di_v0,
          w_v0, rows_v0, isem0, gsem0, False)
    back(e2 - 1, gi_v1, ci_v1, w_v1, rows_v1, di_v1, gsem1)
    return carry

  lax.fori_loop(0, (CPT - 2) // 2, pair, 0)
  front(jnp.int32(CPT - 1), src_v1, dst_v1, typ_v1, gi_v1, ci_v1, di_v1,
        w_v1, rows_v1, isem1, gsem1, True)
  back(jnp.int32(CPT - 2), gi_v0, ci_v0, w_v0, rows_v0, di_v0, gsem0)
  back(jnp.int32(CPT - 1), gi_v1, ci_v1, w_v1, rows_v1, di_v1, gsem1)
  wait_in(jnp.int32(CPT), src_v0, dst_v0, typ_v0, isem0)
  plsc.subcore_barrier()
  pltpu.sync_copy(acc_sh.at[pl.ds(sid * RPT, RPT), :],
                  out_hbm.at[cid, pl.ds(sid * RPT, RPT), :])


def _combine_body(comp_ref, basis_ref, out_ref):
  out_ref[...] = jnp.dot(comp_ref[...], basis_ref[...],
                         preferred_element_type=_f32)


def _combine(comp, basis_flat):
  nb = comp.shape[1]
  f = basis_flat.shape[1]
  blk = 2048
  return pl.pallas_call(
      _combine_body,
      grid=(f // blk,),
      in_specs=[
          pl.BlockSpec((N_REL, nb), lambda i: (0, 0)),
          pl.BlockSpec((nb, blk), lambda i: (0, i)),
      ],
      out_specs=pl.BlockSpec((N_REL, blk), lambda i: (0, i)),
      out_shape=jax.ShapeDtypeStruct((N_REL, f), _f32),
  )(comp, basis_flat)


def _hx_body(x_ref, w_ref, out_ref):
  out_ref[0] = jnp.dot(x_ref[...], w_ref[0], preferred_element_type=_f32)


def _hx(x, w_full):
  bn = 1000
  din = x.shape[1]
  return pl.pallas_call(
      _hx_body,
      grid=(N_NODES // bn, 17),
      in_specs=[
          pl.BlockSpec((bn, din), lambda nb, r: (nb, 0)),
          pl.BlockSpec((1, din, D), lambda nb, r: (r, 0, 0)),
      ],
      out_specs=pl.BlockSpec((1, bn, D), lambda nb, r: (r, nb, 0)),
      out_shape=jax.ShapeDtypeStruct((17, N_NODES, D), _f32),
  )(x, w_full)


def _recip_body(cnt_ref, out_ref):
  c = cnt_ref[0] + cnt_ref[1]
  out_ref[...] = 1.0 / jnp.maximum(c, 1.0)


def _recip(cnt):
  rows = NR_PAD // D
  blk = 128
  out = pl.pallas_call(
      _recip_body,
      grid=(rows // blk,),
      in_specs=[pl.BlockSpec((2, blk, D), lambda i: (0, i, 0))],
      out_specs=pl.BlockSpec((blk, D), lambda i: (i, 0)),
      out_shape=jax.ShapeDtypeStruct((rows, D), _f32),
  )(cnt.reshape(2, rows, D))
  return out.reshape(NR_PAD)


def _z_body(agg_ref, root_ref, bias_ref, out_ref):
  s = agg_ref[0] + agg_ref[1] + root_ref[0] + bias_ref[...]
  out_ref[...] = jnp.where(s >= 0, s, 0.01 * s)


def _z_layer(agg, hx1, bias1):
  bn = 1000
  return pl.pallas_call(
      _z_body,
      grid=(N_NODES // bn,),
      in_specs=[
          pl.BlockSpec((2, bn, D), lambda i: (0, i, 0)),
          pl.BlockSpec((1, bn, D), lambda i: (16, i, 0)),
          pl.BlockSpec((1, D), lambda i: (0, 0)),
      ],
      out_specs=pl.BlockSpec((bn, D), lambda i: (i, 0)),
      out_shape=jax.ShapeDtypeStruct((N_NODES, D), _f32),
  )(agg, hx1, bias1.reshape(1, D))


def _final_body(agg_ref, root_ref, bias_ref, mu_ref, ls_ref):
  s = agg_ref[0] + agg_ref[1] + root_ref[0] + bias_ref[...]
  mu_ref[...] = s[:, :64]
  ls_ref[...] = s[:, 64:]


def _final(agg23, hx23, bias23):
  bn = 1000
  return pl.pallas_call(
      _final_body,
      grid=(N_NODES // bn,),
      in_specs=[
          pl.BlockSpec((2, bn, D), lambda i: (0, i, 0)),
          pl.BlockSpec((1, bn, D), lambda i: (16, i, 0)),
          pl.BlockSpec((1, D), lambda i: (0, 0)),
      ],
      out_specs=[
          pl.BlockSpec((bn, 64), lambda i: (i, 0)),
          pl.BlockSpec((bn, 64), lambda i: (i, 0)),
      ],
      out_shape=(jax.ShapeDtypeStruct((N_NODES, 64), _f32),
                 jax.ShapeDtypeStruct((N_NODES, 64), _f32)),
  )(agg23, hx23, bias23.reshape(1, D))


def kernel(x, edge_index, edge_type, comp1, basis1, root1, bias1,
           comp_mu, basis_mu, root_mu, bias_mu,
           comp_ls, basis_ls, root_ls, bias_ls):
  nb = basis1.shape[0]

  zc = jnp.zeros((NR_PAD,), _f32)
  za = jnp.zeros((N_ACC, D), _f32)
  src_arr = edge_index[0]
  dst_arr = edge_index[1]

  cnt = _counts_kernel(dst_arr, edge_type, zc)
  recip = _recip(cnt)

  w1 = _combine(comp1, basis1.reshape(nb, -1)).reshape(N_REL, D, D)
  w1_full = jnp.concatenate([w1, root1[None]], axis=0)
  hx1 = _hx(x, w1_full)
  agg1 = _agg_kernel(hx1.reshape(17 * N_NODES, D), src_arr, dst_arr,
                     edge_type, recip, za)
  z = _z_layer(agg1, hx1, bias1)

  wmu = _combine(comp_mu, basis_mu.reshape(nb, -1)).reshape(N_REL, D, 64)
  wls = _combine(comp_ls, basis_ls.reshape(nb, -1)).reshape(N_REL, D, 64)
  w23 = jnp.concatenate([wmu, wls], axis=2)
  root23 = jnp.concatenate([root_mu, root_ls], axis=1)
  w23_full = jnp.concatenate([w23, root23[None]], axis=0)
  hx23 = _hx(z, w23_full)
  agg23 = _agg_kernel(hx23.reshape(17 * N_NODES, D), src_arr, dst_arr,
                      edge_type, recip, za)
  bias23 = jnp.concatenate([bias_mu, bias_ls], axis=0)
  mu, logstd = _final(agg23, hx23, bias23)
  return (mu, logstd)
